# Initial kernel scaffold; baseline (speedup 1.0000x reference)
import jax
import jax.numpy as jnp
from jax.experimental import pallas as pl


def kernel(samples):
    raise NotImplementedError("write your pallas kernel here")


# speedup vs baseline: 5515.0279x; 5515.0279x over previous
import jax
import jax.numpy as jnp
from jax import lax
from jax.experimental import pallas as pl
from jax.experimental.pallas import tpu as pltpu
from jax.experimental.pallas import tpu_sc as plsc

N = 4096
B = 16384
D = 128
M = B - N
NC, NS, L = 2, 16, 16
NW = NC * NS
RPW = N // NW
CH = M // L


def _reservoir_body(samples_hbm, idx_hbm, out_hbm, idx_v, src_v, rows_v, sem):
    wid = lax.axis_index("s") * NC + lax.axis_index("c")
    base = wid * RPW
    pltpu.sync_copy(idx_hbm, idx_v)
    lane = lax.iota(jnp.int32, L)

    for g in range(RPW // L):
        src_v[pl.ds(g * L, L)] = base + g * L + lane

    def body(c, carry):
        iv = idx_v[pl.ds(c * L, L)]
        tv = N + c * L + lane
        rel = iv - base
        valid = (rel >= 0) & (rel < RPW)
        key = jnp.where(valid, rel * L, RPW * L) + lane
        k_s, t_s = plsc.sort_key_val(key, tv)
        rel_s = lax.shift_right_arithmetic(k_s, 4)
        valid_s = k_s < RPW * L
        nxt = lax.gather(
            k_s,
            jnp.minimum(lane + 1, L - 1)[:, None],
            lax.GatherDimensionNumbers(
                offset_dims=(), collapsed_slice_dims=(0,), start_index_map=(0,)
            ),
            slice_sizes=(1,),
            mode=lax.GatherScatterMode.PROMISE_IN_BOUNDS,
        )
        winner = (rel_s != lax.shift_right_arithmetic(nxt, 4)) | (lane == L - 1)
        mask = winner & valid_s
        plsc.store_scatter(src_v, [jnp.where(valid_s, rel_s, 0)], t_s, mask=mask)
        return carry

    lax.fori_loop(0, CH, body, 0)

    pltpu.async_copy(samples_hbm.at[src_v], rows_v, sem).wait()
    pltpu.sync_copy(rows_v, out_hbm.at[pl.ds(base, RPW)])


def kernel(samples):
    samples = lax.stop_gradient(samples)
    rng = jax.random.key(42)
    t = jnp.arange(M)
    keys = jax.vmap(lambda tt: jax.random.fold_in(rng, tt))(t)
    idx = jax.vmap(lambda k, mx: jax.random.randint(k, (), 0, mx))(keys, N + t + 1)
    idx = idx.astype(jnp.int32)

    mesh = plsc.VectorSubcoreMesh(
        core_axis_name="c", subcore_axis_name="s", num_cores=NC, num_subcores=NS
    )
    run = pl.kernel(
        _reservoir_body,
        out_type=jax.ShapeDtypeStruct((N, D), jnp.float32),
        mesh=mesh,
        compiler_params=pltpu.CompilerParams(needs_layout_passes=False),
        scratch_types=[
            pltpu.VMEM((M,), jnp.int32),
            pltpu.VMEM((RPW,), jnp.int32),
            pltpu.VMEM((RPW, D), jnp.float32),
            pltpu.SemaphoreType.DMA,
        ],
    )
    return run(samples, idx)


# speedup vs baseline: 9012.7904x; 1.6342x over previous
import jax
import jax.numpy as jnp
from jax import lax
from jax.experimental import pallas as pl
from jax.experimental.pallas import tpu as pltpu
from jax.experimental.pallas import tpu_sc as plsc

N = 4096
B = 16384
D = 128
M = B - N
NC, NS, L = 2, 16, 16
HALF = N // NC
RPW = HALF // NS
TPW = M // NS
CH = TPW // L


def _reservoir_body(
    samples_hbm, idx_hbm, out_hbm, idx_v, loc_v, stripe_v, src_v, rows_v, shared, sem
):
    c = lax.axis_index("c")
    s = lax.axis_index("s")
    half_base = c * HALF
    base = half_base + s * RPW
    lane = lax.iota(jnp.int32, L)

    pltpu.sync_copy(idx_hbm.at[pl.ds(s * TPW, TPW)], idx_v)

    def init(g, carry):
        loc_v[pl.ds(g * L, L)] = half_base + g * L + lane
        return carry

    lax.fori_loop(0, HALF // L, init, 0)

    def body(k, carry):
        iv = idx_v[pl.ds(k * L, L)]
        tv = N + s * TPW + k * L + lane
        rel = iv - half_base
        valid = (rel >= 0) & (rel < HALF)
        key = jnp.where(valid, rel * L, HALF * L) + lane
        k_s, t_s = plsc.sort_key_val(key, tv)
        rel_s = lax.shift_right_arithmetic(k_s, 4)
        valid_s = k_s < HALF * L
        nxt = lax.gather(
            k_s,
            jnp.minimum(lane + 1, L - 1)[:, None],
            lax.GatherDimensionNumbers(
                offset_dims=(), collapsed_slice_dims=(0,), start_index_map=(0,)
            ),
            slice_sizes=(1,),
            mode=lax.GatherScatterMode.PROMISE_IN_BOUNDS,
        )
        winner = (rel_s != lax.shift_right_arithmetic(nxt, 4)) | (lane == L - 1)
        mask = winner & valid_s
        plsc.store_scatter(loc_v, [jnp.where(valid_s, rel_s, 0)], t_s, mask=mask)
        return carry

    lax.fori_loop(0, CH, body, 0)

    pltpu.sync_copy(loc_v, shared.at[s])
    plsc.subcore_barrier()
    pltpu.sync_copy(shared.at[:, pl.ds(s * RPW, RPW)], stripe_v)

    for g in range(RPW // L):
        acc = stripe_v[0, pl.ds(g * L, L)]
        for r in range(1, NS):
            acc = jnp.maximum(acc, stripe_v[r, pl.ds(g * L, L)])
        src_v[pl.ds(g * L, L)] = acc

    pltpu.async_copy(samples_hbm.at[src_v], rows_v, sem).wait()
    pltpu.sync_copy(rows_v, out_hbm.at[pl.ds(base, RPW)])


def kernel(samples):
    samples = lax.stop_gradient(samples)
    rng = jax.random.key(42)
    t = jnp.arange(M)
    keys = jax.vmap(lambda tt: jax.random.fold_in(rng, tt))(t)
    idx = jax.vmap(lambda k, mx: jax.random.randint(k, (), 0, mx))(keys, N + t + 1)
    idx = idx.astype(jnp.int32)

    mesh = plsc.VectorSubcoreMesh(
        core_axis_name="c", subcore_axis_name="s", num_cores=NC, num_subcores=NS
    )
    run = pl.kernel(
        _reservoir_body,
        out_type=jax.ShapeDtypeStruct((N, D), jnp.float32),
        mesh=mesh,
        compiler_params=pltpu.CompilerParams(needs_layout_passes=False),
        scratch_types=[
            pltpu.VMEM((TPW,), jnp.int32),
            pltpu.VMEM((HALF,), jnp.int32),
            pltpu.VMEM((NS, RPW), jnp.int32),
            pltpu.VMEM((RPW,), jnp.int32),
            pltpu.VMEM((RPW, D), jnp.float32),
            pltpu.VMEM_SHARED((NS, HALF), jnp.int32),
            pltpu.SemaphoreType.DMA,
        ],
    )
    return run(samples, idx)
